# baseline (speedup 1.0000x reference)
import functools

import jax
import jax.numpy as jnp
from jax import lax
from jax.experimental import pallas as pl
from jax.experimental.pallas import tpu as pltpu
from jax.experimental.pallas import tpu_sc as plsc

NC = 2
NS = 16
NW = NC * NS
LANES = 16
CH = 128


def _prep_body(seg_ids_ref, pos_ref, seg_tab_ref, comb_ref, cidx_ref):
    comb_ref[...] = pos_ref[...][:, None, :] + seg_tab_ref[...][None, :, :]
    s_iota = lax.broadcasted_iota(jnp.int32, seg_ids_ref.shape, 1)
    cidx_ref[...] = seg_ids_ref[...] + 3 * s_iota


def _sc_body(n_chunks, tok_hbm, ids_hbm, cidx_hbm, comb_hbm, out_hbm,
             idx_t, idx_c, rt0, rt1, rt2, ro0, ro1, ro2,
             gs0, gs1, gs2, ss0, ss1, ss2):
    wid = lax.axis_index("s") * NC + lax.axis_index("c")
    pw = n_chunks * CH
    base0 = wid * pw

    pltpu.sync_copy(ids_hbm.at[pl.ds(base0, pw)], idx_t)
    pltpu.sync_copy(cidx_hbm.at[pl.ds(base0, pw)], idx_c)

    def fire(g, rt, ro, gsem):
        it = idx_t.at[pl.ds(g * CH, CH)]
        ic = idx_c.at[pl.ds(g * CH, CH)]
        pltpu.async_copy(tok_hbm.at[it], rt, gsem)
        pltpu.async_copy(comb_hbm.at[ic], ro, gsem)

    def wait_scatter(rt, ssem):
        pltpu.make_async_copy(rt, out_hbm.at[pl.ds(base0, CH)], ssem).wait()

    def proc(g, rt, ro, gsem, ssem):
        dummy = tok_hbm.at[pl.ds(0, CH)]
        pltpu.make_async_copy(dummy, rt, gsem).wait()
        pltpu.make_async_copy(dummy, ro, gsem).wait()

        def row_body(r, rcarry):
            for c4 in range(4):
                w = ro[r, pl.ds(c4 * LANES, LANES)]
                a16 = lax.bitcast_convert_type(w << 16, jnp.float32)
                b16 = lax.bitcast_convert_type(
                    w & jnp.int32(-65536), jnp.float32)
                plsc.addupdate(rt.at[r, pl.ds(c4 * 2 * LANES, LANES)], a16)
                plsc.addupdate(
                    rt.at[r, pl.ds(c4 * 2 * LANES + LANES, LANES)], b16)
            return rcarry

        lax.fori_loop(0, CH, row_body, 0)
        pltpu.async_copy(rt, out_hbm.at[pl.ds(base0 + g * CH, CH)], ssem)

    slots = ((rt0, ro0, gs0, ss0), (rt1, ro1, gs1, ss1), (rt2, ro2, gs2, ss2))
    nslots = len(slots)
    n_main = n_chunks // nslots
    n_tail = n_chunks - n_main * nslots

    for i, (rt, ro, gsem, _) in enumerate(slots):
        fire(i, rt, ro, gsem)

    def round_body(k, carry):
        g0 = nslots * k
        for i, (rt, ro, gsem, ssem) in enumerate(slots):
            proc(g0 + i, rt, ro, gsem, ssem)
            gn = g0 + i + nslots

            @pl.when(gn < n_chunks)
            def _(rt=rt, ro=ro, gsem=gsem, ssem=ssem, gn=gn):
                wait_scatter(rt, ssem)
                fire(gn, rt, ro, gsem)

        return carry

    lax.fori_loop(0, n_main, round_body, 0)
    for i in range(n_tail):
        rt, ro, gsem, ssem = slots[i]
        proc(n_main * nslots + i, rt, ro, gsem, ssem)
    for rt, _, _, ssem in slots:
        wait_scatter(rt, ssem)


def kernel(input_ids, segment_ids, token_table, segment_table, pos_table):
    B, S = input_ids.shape
    H = token_table.shape[1]
    R = B * S
    assert R % (NW * CH) == 0
    n_chunks = R // (NW * CH)

    comb3, cidx = pl.pallas_call(
        _prep_body,
        out_shape=(
            jax.ShapeDtypeStruct((S, 3, H), jnp.float32),
            jax.ShapeDtypeStruct((B, S), jnp.int32),
        ),
    )(segment_ids.astype(jnp.int32), pos_table[:S], segment_table)

    comb = comb3.reshape(S * 3, H)
    cg = comb.reshape(S * 3, H // 32, 2, LANES).astype(jnp.bfloat16)
    u16 = lax.bitcast_convert_type(cg, jnp.uint16)
    word = u16[:, :, 0, :].astype(jnp.uint32) | (
        u16[:, :, 1, :].astype(jnp.uint32) << 16)
    comb_bf = lax.bitcast_convert_type(word.reshape(S * 3, H // 2), jnp.int32)

    ids_flat = input_ids.astype(jnp.int32).reshape(R)
    cidx_flat = cidx.reshape(R)

    sc_fn = functools.partial(
        pl.kernel,
        out_type=jax.ShapeDtypeStruct((R, H), jnp.float32),
        mesh=plsc.VectorSubcoreMesh(core_axis_name="c", subcore_axis_name="s"),
        compiler_params=pltpu.CompilerParams(use_tc_tiling_on_sc=False),
        scratch_types=[
            pltpu.VMEM((R // NW,), jnp.int32),
            pltpu.VMEM((R // NW,), jnp.int32),
            pltpu.VMEM((CH, H), jnp.float32),
            pltpu.VMEM((CH, H), jnp.float32),
            pltpu.VMEM((CH, H), jnp.float32),
            pltpu.VMEM((CH, H // 2), jnp.int32),
            pltpu.VMEM((CH, H // 2), jnp.int32),
            pltpu.VMEM((CH, H // 2), jnp.int32),
            pltpu.SemaphoreType.DMA,
            pltpu.SemaphoreType.DMA,
            pltpu.SemaphoreType.DMA,
            pltpu.SemaphoreType.DMA,
            pltpu.SemaphoreType.DMA,
            pltpu.SemaphoreType.DMA,
        ],
    )(functools.partial(_sc_body, n_chunks))

    out2d = sc_fn(token_table, ids_flat, cidx_flat, comb_bf)
    return out2d.reshape(B, S, H)
